# Initial kernel scaffold; baseline (speedup 1.0000x reference)
import jax
import jax.numpy as jnp
from jax.experimental import pallas as pl


def kernel(z, edge_index, batch_size):
    raise NotImplementedError("write your pallas kernel here")


# speedup vs baseline: 1.0339x; 1.0339x over previous
import functools

import jax
import jax.numpy as jnp
from jax import lax
from jax.experimental import pallas as pl
from jax.experimental.pallas import tpu as pltpu
from jax.experimental.pallas import tpu_sc as plsc

_SPREAD_W = 0.1
_QUAD_W = 1.0
_EPS = 1e-9
_CH = 80


@functools.lru_cache(maxsize=None)
def _build_sc_call(N, Fp, F, TOTP, per_tile, NC, NS):
    L = 16
    niter = per_tile // _CH
    ngrp = _CH // L
    mesh = plsc.VectorSubcoreMesh(core_axis_name="c", subcore_axis_name="s")
    nw = NC * NS

    @functools.partial(
        pl.kernel,
        mesh=mesh,
        compiler_params=pltpu.CompilerParams(
            use_tc_tiling_on_sc=False, needs_layout_passes=False),
        out_type=jax.ShapeDtypeStruct((3 * nw, L), jnp.float32),
        scratch_types=[
            pltpu.VMEM((_CH,), jnp.int32),
            pltpu.VMEM((_CH,), jnp.int32),
            pltpu.VMEM((_CH,), jnp.float32),
            pltpu.VMEM((_CH,), jnp.float32),
            pltpu.VMEM((_CH, Fp), jnp.float32),
            pltpu.VMEM((_CH, Fp), jnp.float32),
            pltpu.VMEM((L,), jnp.float32),
            pltpu.SemaphoreType.DMA,
        ],
    )
    def sc_call(z_hbm, w_hbm, sidx_hbm, didx_hbm, wp_hbm, wn_hbm, out_hbm,
                sidx_v, didx_v, wp_v, wn_v, srows, drows, accbuf, sem):
        wid = lax.axis_index("s") * NC + lax.axis_index("c")
        base = wid * per_tile
        zf16 = jnp.zeros((L,), jnp.float32)

        def outer(it, carry):
            pacc, nacc, wacc = carry
            off = base + it * _CH
            pltpu.sync_copy(sidx_hbm.at[pl.ds(off, _CH)], sidx_v)
            pltpu.sync_copy(didx_hbm.at[pl.ds(off, _CH)], didx_v)
            pltpu.sync_copy(wp_hbm.at[pl.ds(off, _CH)], wp_v)
            pltpu.sync_copy(wn_hbm.at[pl.ds(off, _CH)], wn_v)
            cp1 = pltpu.async_copy(z_hbm.at[sidx_v], srows, sem)
            cp2 = pltpu.async_copy(w_hbm.at[didx_v], drows, sem)
            cp1.wait()
            cp2.wait()
            for g in range(ngrp):
                rows = lax.iota(jnp.int32, L) + (g * L)

                def kbody(_, kc):
                    colv, cr, daa, dbb = kc
                    va = plsc.load_gather(srows, [rows, colv])
                    vb = plsc.load_gather(drows, [rows, colv])
                    return (colv + 1, cr + va * vb,
                            daa + va * va, dbb + vb * vb)

                init = (jnp.zeros((L,), jnp.int32), zf16, zf16, zf16)
                _, cr, daa, dbb = lax.fori_loop(0, F - 1, kbody, init,
                                                unroll=8)
                ct = jnp.full((L,), F - 1, jnp.int32)
                va6 = plsc.load_gather(srows, [rows, ct])
                vb6 = plsc.load_gather(drows, [rows, ct])
                cr = cr + va6 * vb6
                daa = daa - va6 * va6
                dbb = dbb - vb6 * vb6
                q = 1.0 - (cr * cr) / (daa * dbb + _EPS)
                qm = jnp.minimum(q, 10.0)
                wpv = plsc.load_gather(wp_v, [rows])
                wnv = plsc.load_gather(wn_v, [rows])
                pacc = pacc + wpv * qm
                nacc = nacc + wnv * jnp.maximum(1.0 - qm, 0.0)
                wacc = wacc + wpv
            return (pacc, nacc, wacc)

        pacc, nacc, wacc = lax.fori_loop(0, niter, outer,
                                         (zf16, zf16, zf16))
        accbuf[...] = pacc
        pltpu.sync_copy(accbuf, out_hbm.at[wid])
        accbuf[...] = nacc
        pltpu.sync_copy(accbuf, out_hbm.at[nw + wid])
        accbuf[...] = wacc
        pltpu.sync_copy(accbuf, out_hbm.at[2 * nw + wid])

    return sc_call, nw


def kernel(z, edge_index, batch_size):
    z = z.astype(jnp.float32)
    ei = edge_index.astype(jnp.int32)
    N, F = z.shape
    E = ei.shape[1]
    n_neg = N

    info = plsc.get_sparse_core_info()
    NC, NS, L = info.num_cores, info.num_subcores, info.num_lanes
    nw = NC * NS

    neg = jax.random.randint(jax.random.key(42), (2, n_neg), 0, batch_size)
    neg = neg.astype(jnp.int32)

    Fp = ((F + L - 1) // L) * L
    zpad = jnp.zeros((N, Fp), jnp.float32).at[:, :F].set(z)
    sign = jnp.ones((Fp,), jnp.float32).at[F - 1].set(-1.0)
    wpad = zpad * sign

    TOT = E + n_neg
    per_tile = -(-TOT // (nw * _CH)) * _CH
    TOTP = per_tile * nw
    pad = TOTP - TOT
    src_all = jnp.concatenate(
        [ei[0], neg[0], jnp.zeros((pad,), jnp.int32)])
    dst_all = jnp.concatenate(
        [ei[1], neg[1], jnp.zeros((pad,), jnp.int32)])
    valid = ((ei[0] < batch_size) & (ei[1] < batch_size)).astype(jnp.float32)
    wp_all = jnp.concatenate([valid, jnp.zeros((n_neg + pad,), jnp.float32)])
    wn_all = jnp.concatenate(
        [jnp.zeros((E,), jnp.float32), jnp.ones((n_neg,), jnp.float32),
         jnp.zeros((pad,), jnp.float32)])

    sc_call, nw = _build_sc_call(N, Fp, F, TOTP, per_tile, NC, NS)
    parts = sc_call(zpad, wpad, src_all, dst_all, wp_all, wn_all)

    pos_sum = jnp.sum(parts[0:nw])
    neg_sum = jnp.sum(parts[nw:2 * nw])
    mask_count = jnp.sum(parts[2 * nw:3 * nw])
    pos_loss = pos_sum / mask_count
    neg_loss = neg_sum / n_neg
    spread_loss = _SPREAD_W * pos_sum / mask_count
    total = _QUAD_W * (pos_loss + neg_loss) + spread_loss
    return jnp.clip(total, 0.0, 100.0)


# speedup vs baseline: 1.4308x; 1.3839x over previous
import functools

import jax
import jax.numpy as jnp
from jax import lax
from jax.experimental import pallas as pl
from jax.experimental.pallas import tpu as pltpu
from jax.experimental.pallas import tpu_sc as plsc

_SPREAD_W = 0.1
_QUAD_W = 1.0
_EPS = 1e-9
_CH = 80


def _d_body(z_ref, w_ref, o_ref):
    o_ref[...] = jnp.sum(z_ref[...] * w_ref[...], axis=1, keepdims=True)


@functools.lru_cache(maxsize=None)
def _build_d_call(N, Fp):
    nblk = 1
    for cand in (10, 8, 5, 4, 2):
        if N % cand == 0 and (N // cand) % 8 == 0:
            nblk = cand
            break
    br = N // nblk
    return pl.pallas_call(
        _d_body,
        grid=(nblk,),
        in_specs=[pl.BlockSpec((br, Fp), lambda i: (i, 0)),
                  pl.BlockSpec((br, Fp), lambda i: (i, 0))],
        out_specs=pl.BlockSpec((br, 1), lambda i: (i, 0)),
        out_shape=jax.ShapeDtypeStruct((N, 1), jnp.float32),
    )


@functools.lru_cache(maxsize=None)
def _build_sc_call(N, Fp, per_tile, NC, NS):
    L = 16
    niter = per_tile // _CH
    ngrp = _CH // L
    nkb = Fp // L
    mesh = plsc.VectorSubcoreMesh(core_axis_name="c", subcore_axis_name="s")
    nw = NC * NS

    @functools.partial(
        pl.kernel,
        mesh=mesh,
        compiler_params=pltpu.CompilerParams(
            use_tc_tiling_on_sc=False, needs_layout_passes=False),
        out_type=jax.ShapeDtypeStruct((3 * nw, L), jnp.float32),
        scratch_types=[
            pltpu.VMEM((N,), jnp.float32),
            pltpu.VMEM((4 * _CH,), jnp.int32),
            pltpu.VMEM((4 * _CH,), jnp.int32),
            pltpu.VMEM((_CH, Fp), jnp.float32),
            pltpu.VMEM((_CH, Fp), jnp.float32),
            pltpu.VMEM((_CH, Fp), jnp.float32),
            pltpu.VMEM((_CH, Fp), jnp.float32),
            pltpu.VMEM((L,), jnp.float32),
            pltpu.SemaphoreType.DMA,
            pltpu.SemaphoreType.DMA,
        ],
    )
    def sc_call(z_hbm, w_hbm, d_hbm, meta_hbm, out_hbm,
                d_v, meta0, meta1, s0, s1, t0, t1, accbuf, semA, semB):
        wid = lax.axis_index("s") * NC + lax.axis_index("c")
        mbase = wid * niter
        zf16 = jnp.zeros((L,), jnp.float32)
        zi16 = jnp.zeros((L,), jnp.int32)
        metas = (meta0, meta1)
        srcs = (s0, s1)
        dsts = (t0, t1)
        sems = (semA, semB)

        def fire_rows(b):
            m = metas[b]
            pltpu.async_copy(z_hbm.at[m.at[pl.ds(0, _CH)]], srcs[b], sems[b])
            pltpu.async_copy(w_hbm.at[m.at[pl.ds(_CH, _CH)]], dsts[b],
                             sems[b])

        def wait_rows(b):
            pltpu.make_async_copy(z_hbm.at[pl.ds(0, _CH)], srcs[b],
                                  sems[b]).wait()
            pltpu.make_async_copy(z_hbm.at[pl.ds(0, _CH)], dsts[b],
                                  sems[b]).wait()

        def compute(b, pacc, nacc, wacc):
            m = metas[b]
            sb = srcs[b]
            tb = dsts[b]
            for g in range(ngrp):
                rows = lax.iota(jnp.int32, L) + (g * L)
                vsi = plsc.load_gather(m, [rows])
                vdi = plsc.load_gather(m, [rows + _CH])
                wpv = plsc.bitcast(plsc.load_gather(m, [rows + 2 * _CH]),
                                   jnp.float32)
                wnv = plsc.bitcast(plsc.load_gather(m, [rows + 3 * _CH]),
                                   jnp.float32)
                dai = plsc.load_gather(d_v, [vsi])
                dbj = plsc.load_gather(d_v, [vdi])

                def kb(_, kc):
                    colv, cr = kc
                    for cc in range(L):
                        col = colv + cc
                        va = plsc.load_gather(sb, [rows, col])
                        vb = plsc.load_gather(tb, [rows, col])
                        cr = cr + va * vb
                    return (colv + L, cr)

                _, cr = lax.fori_loop(0, nkb, kb, (zi16, zf16))
                q = 1.0 - (cr * cr) / (dai * dbj + _EPS)
                qm = jnp.minimum(q, 10.0)
                pacc = pacc + wpv * qm
                nacc = nacc + wnv * jnp.maximum(1.0 - qm, 0.0)
                wacc = wacc + wpv
            return pacc, nacc, wacc

        def segment(b, i, carry):
            pacc, nacc, wacc = carry
            inext = jnp.minimum(i + 1, niter - 1)
            pltpu.sync_copy(meta_hbm.at[mbase + inext], metas[1 - b])
            fire_rows(1 - b)
            wait_rows(b)
            return compute(b, pacc, nacc, wacc)

        pltpu.sync_copy(d_hbm, d_v)
        pltpu.sync_copy(meta_hbm.at[mbase], meta0)
        fire_rows(0)

        def body2(j, carry):
            carry = segment(0, 2 * j, carry)
            carry = segment(1, 2 * j + 1, carry)
            return carry

        carry = lax.fori_loop(0, niter // 2, body2, (zf16, zf16, zf16))
        if niter % 2:
            carry = segment(0, niter - 1, carry)
            wait_rows(1)
        else:
            wait_rows(0)
        pacc, nacc, wacc = carry
        accbuf[...] = pacc
        pltpu.sync_copy(accbuf, out_hbm.at[wid])
        accbuf[...] = nacc
        pltpu.sync_copy(accbuf, out_hbm.at[nw + wid])
        accbuf[...] = wacc
        pltpu.sync_copy(accbuf, out_hbm.at[2 * nw + wid])

    return sc_call, nw


def kernel(z, edge_index, batch_size):
    z = z.astype(jnp.float32)
    ei = edge_index.astype(jnp.int32)
    N, F = z.shape
    E = ei.shape[1]
    n_neg = N

    info = plsc.get_sparse_core_info()
    NC, NS, L = info.num_cores, info.num_subcores, info.num_lanes
    nw = NC * NS

    neg = jax.random.randint(jax.random.key(42), (2, n_neg), 0, batch_size)
    neg = neg.astype(jnp.int32)

    Fp = ((F + L - 1) // L) * L
    zpad = jnp.zeros((N, Fp), jnp.float32).at[:, :F].set(z)
    sign = jnp.ones((Fp,), jnp.float32).at[F - 1].set(-1.0)
    wpad = zpad * sign

    d = _build_d_call(N, Fp)(zpad, wpad)
    d = d[:, 0]

    TOT = E + n_neg
    per_tile = -(-TOT // (nw * _CH)) * _CH
    TOTP = per_tile * nw
    niter = per_tile // _CH
    pad = TOTP - TOT
    src_all = jnp.concatenate(
        [ei[0], neg[0], jnp.zeros((pad,), jnp.int32)])
    dst_all = jnp.concatenate(
        [ei[1], neg[1], jnp.zeros((pad,), jnp.int32)])
    valid = ((ei[0] < batch_size) & (ei[1] < batch_size)).astype(jnp.float32)
    wp_all = jnp.concatenate([valid, jnp.zeros((n_neg + pad,), jnp.float32)])
    wn_all = jnp.concatenate(
        [jnp.zeros((E,), jnp.float32), jnp.ones((n_neg,), jnp.float32),
         jnp.zeros((pad,), jnp.float32)])
    meta = jnp.concatenate([
        src_all.reshape(nw, niter, _CH),
        dst_all.reshape(nw, niter, _CH),
        lax.bitcast_convert_type(wp_all, jnp.int32).reshape(nw, niter, _CH),
        lax.bitcast_convert_type(wn_all, jnp.int32).reshape(nw, niter, _CH),
    ], axis=-1).reshape(nw * niter, 4 * _CH)

    sc_call, nw = _build_sc_call(N, Fp, per_tile, NC, NS)
    parts = sc_call(zpad, wpad, d, meta)

    pos_sum = jnp.sum(parts[0:nw])
    neg_sum = jnp.sum(parts[nw:2 * nw])
    mask_count = jnp.sum(parts[2 * nw:3 * nw])
    pos_loss = pos_sum / mask_count
    neg_loss = neg_sum / n_neg
    spread_loss = _SPREAD_W * pos_sum / mask_count
    total = _QUAD_W * (pos_loss + neg_loss) + spread_loss
    return jnp.clip(total, 0.0, 100.0)


# speedup vs baseline: 2.6920x; 1.8814x over previous
import functools

import jax
import jax.numpy as jnp
from jax import lax
from jax.experimental import pallas as pl
from jax.experimental.pallas import tpu as pltpu
from jax.experimental.pallas import tpu_sc as plsc

_SPREAD_W = 0.1
_QUAD_W = 1.0
_EPS = 1e-9
_CH = 80


def _d_body(z_ref, w_ref, o_ref):
    o_ref[...] = jnp.sum(z_ref[...] * w_ref[...], axis=1, keepdims=True)


@functools.lru_cache(maxsize=None)
def _build_d_call(N, Fp):
    nblk = 1
    for cand in (10, 8, 5, 4, 2):
        if N % cand == 0 and (N // cand) % 8 == 0:
            nblk = cand
            break
    br = N // nblk
    return pl.pallas_call(
        _d_body,
        grid=(nblk,),
        in_specs=[pl.BlockSpec((br, Fp), lambda i: (i, 0)),
                  pl.BlockSpec((br, Fp), lambda i: (i, 0))],
        out_specs=pl.BlockSpec((br, 1), lambda i: (i, 0)),
        out_shape=jax.ShapeDtypeStruct((N, 1), jnp.float32),
    )


@functools.lru_cache(maxsize=None)
def _build_sc_call(N, Fp, per_tile, NC, NS):
    L = 16
    niter = per_tile // _CH
    ngrp = _CH // L
    nkb = Fp // L
    mesh = plsc.VectorSubcoreMesh(core_axis_name="c", subcore_axis_name="s")
    nw = NC * NS

    @functools.partial(
        pl.kernel,
        mesh=mesh,
        compiler_params=pltpu.CompilerParams(
            use_tc_tiling_on_sc=False, needs_layout_passes=False),
        out_type=jax.ShapeDtypeStruct((3 * nw, L), jnp.float32),
        scratch_types=[
            pltpu.VMEM((N,), jnp.float32),
            pltpu.VMEM((4 * _CH,), jnp.int32),
            pltpu.VMEM((4 * _CH,), jnp.int32),
            pltpu.VMEM((_CH, Fp), jnp.float32),
            pltpu.VMEM((_CH, Fp), jnp.float32),
            pltpu.VMEM((_CH, Fp), jnp.float32),
            pltpu.VMEM((_CH, Fp), jnp.float32),
            pltpu.VMEM((L,), jnp.float32),
            pltpu.SemaphoreType.DMA,
            pltpu.SemaphoreType.DMA,
        ],
    )
    def sc_call(z_hbm, w_hbm, d_hbm, meta_hbm, out_hbm,
                d_v, meta0, meta1, s0, s1, t0, t1, accbuf, semA, semB):
        wid = lax.axis_index("s") * NC + lax.axis_index("c")
        mbase = wid * niter
        zf16 = jnp.zeros((L,), jnp.float32)
        zi16 = jnp.zeros((L,), jnp.int32)
        metas = (meta0, meta1)
        srcs = (s0, s1)
        dsts = (t0, t1)
        sems = (semA, semB)

        def fire_rows(b):
            m = metas[b]
            pltpu.async_copy(z_hbm.at[m.at[pl.ds(0, _CH)]], srcs[b], sems[b])
            pltpu.async_copy(w_hbm.at[m.at[pl.ds(_CH, _CH)]], dsts[b],
                             sems[b])

        def wait_rows(b):
            pltpu.make_async_copy(z_hbm.at[pl.ds(0, _CH)], srcs[b],
                                  sems[b]).wait()
            pltpu.make_async_copy(z_hbm.at[pl.ds(0, _CH)], dsts[b],
                                  sems[b]).wait()

        def compute(b, pacc, nacc, wacc):
            m = metas[b]
            sb = srcs[b]
            tb = dsts[b]
            for g in range(ngrp):
                rows = lax.iota(jnp.int32, L) + (g * L)
                vsi = plsc.load_gather(m, [rows])
                vdi = plsc.load_gather(m, [rows + _CH])
                wpv = plsc.bitcast(plsc.load_gather(m, [rows + 2 * _CH]),
                                   jnp.float32)
                wnv = plsc.bitcast(plsc.load_gather(m, [rows + 3 * _CH]),
                                   jnp.float32)
                dai = plsc.load_gather(d_v, [vsi])
                dbj = plsc.load_gather(d_v, [vdi])

                lane = lax.iota(jnp.int32, L)

                def kb(_, kc):
                    colv, cr = kc
                    for cc in range(L):
                        col = colv + ((lane + cc) & (L - 1))
                        va = plsc.load_gather(sb, [rows, col])
                        vb = plsc.load_gather(tb, [rows, col])
                        cr = cr + va * vb
                    return (colv + L, cr)

                _, cr = lax.fori_loop(0, nkb, kb, (zi16, zf16))
                q = 1.0 - (cr * cr) / (dai * dbj + _EPS)
                qm = jnp.minimum(q, 10.0)
                pacc = pacc + wpv * qm
                nacc = nacc + wnv * jnp.maximum(1.0 - qm, 0.0)
                wacc = wacc + wpv
            return pacc, nacc, wacc

        def segment(b, i, carry):
            pacc, nacc, wacc = carry
            inext = jnp.minimum(i + 1, niter - 1)
            pltpu.sync_copy(meta_hbm.at[mbase + inext], metas[1 - b])
            fire_rows(1 - b)
            wait_rows(b)
            return compute(b, pacc, nacc, wacc)

        pltpu.sync_copy(d_hbm, d_v)
        pltpu.sync_copy(meta_hbm.at[mbase], meta0)
        fire_rows(0)

        def body2(j, carry):
            carry = segment(0, 2 * j, carry)
            carry = segment(1, 2 * j + 1, carry)
            return carry

        carry = lax.fori_loop(0, niter // 2, body2, (zf16, zf16, zf16))
        if niter % 2:
            carry = segment(0, niter - 1, carry)
            wait_rows(1)
        else:
            wait_rows(0)
        pacc, nacc, wacc = carry
        accbuf[...] = pacc
        pltpu.sync_copy(accbuf, out_hbm.at[wid])
        accbuf[...] = nacc
        pltpu.sync_copy(accbuf, out_hbm.at[nw + wid])
        accbuf[...] = wacc
        pltpu.sync_copy(accbuf, out_hbm.at[2 * nw + wid])

    return sc_call, nw


def kernel(z, edge_index, batch_size):
    z = z.astype(jnp.float32)
    ei = edge_index.astype(jnp.int32)
    N, F = z.shape
    E = ei.shape[1]
    n_neg = N

    info = plsc.get_sparse_core_info()
    NC, NS, L = info.num_cores, info.num_subcores, info.num_lanes
    nw = NC * NS

    neg = jax.random.randint(jax.random.key(42), (2, n_neg), 0, batch_size)
    neg = neg.astype(jnp.int32)

    Fp = ((F + L - 1) // L) * L
    zpad = jnp.zeros((N, Fp), jnp.float32).at[:, :F].set(z)
    sign = jnp.ones((Fp,), jnp.float32).at[F - 1].set(-1.0)
    wpad = zpad * sign

    d = _build_d_call(N, Fp)(zpad, wpad)
    d = d[:, 0]

    TOT = E + n_neg
    per_tile = -(-TOT // (nw * _CH)) * _CH
    TOTP = per_tile * nw
    niter = per_tile // _CH
    pad = TOTP - TOT
    src_all = jnp.concatenate(
        [ei[0], neg[0], jnp.zeros((pad,), jnp.int32)])
    dst_all = jnp.concatenate(
        [ei[1], neg[1], jnp.zeros((pad,), jnp.int32)])
    valid = ((ei[0] < batch_size) & (ei[1] < batch_size)).astype(jnp.float32)
    wp_all = jnp.concatenate([valid, jnp.zeros((n_neg + pad,), jnp.float32)])
    wn_all = jnp.concatenate(
        [jnp.zeros((E,), jnp.float32), jnp.ones((n_neg,), jnp.float32),
         jnp.zeros((pad,), jnp.float32)])
    meta = jnp.concatenate([
        src_all.reshape(nw, niter, _CH),
        dst_all.reshape(nw, niter, _CH),
        lax.bitcast_convert_type(wp_all, jnp.int32).reshape(nw, niter, _CH),
        lax.bitcast_convert_type(wn_all, jnp.int32).reshape(nw, niter, _CH),
    ], axis=-1).reshape(nw * niter, 4 * _CH)

    sc_call, nw = _build_sc_call(N, Fp, per_tile, NC, NS)
    parts = sc_call(zpad, wpad, d, meta)

    pos_sum = jnp.sum(parts[0:nw])
    neg_sum = jnp.sum(parts[nw:2 * nw])
    mask_count = jnp.sum(parts[2 * nw:3 * nw])
    pos_loss = pos_sum / mask_count
    neg_loss = neg_sum / n_neg
    spread_loss = _SPREAD_W * pos_sum / mask_count
    total = _QUAD_W * (pos_loss + neg_loss) + spread_loss
    return jnp.clip(total, 0.0, 100.0)
